# baseline (speedup 1.0000x reference)
import functools

import jax
import jax.numpy as jnp
from jax import lax
from jax.experimental import pallas as pl
from jax.experimental.pallas import tpu as pltpu
from jax.experimental.pallas import tpu_sc as plsc

_H = 512
_W = 512
_N = 128
_K1 = 21.7
_K2 = 40.0
_BIG = 1.0e4
_LN2 = 0.6931471805599453


def _road_loss_tc(hd_ref, pred_ref, out_ref):
    hd = hd_ref[:]
    p0 = pred_ref[:, 0:1]
    p1 = pred_ref[:, 1:2]

    hdn = jnp.concatenate([hd[1:, :], hd[511:, :]], axis=0)
    e = jnp.where(hd != hdn, 0.0, _BIG)

    a = e
    b = jnp.concatenate([jnp.full((1, _W), _BIG, jnp.float32),
                         e[:-1, :]], axis=0)
    s = 1
    for _ in range(9):
        pad = jnp.full((s, _W), _BIG, dtype=jnp.float32)
        a = jnp.minimum(a, jnp.concatenate([a[s:, :], pad], axis=0) + float(s))
        b = jnp.minimum(b, jnp.concatenate([pad, b[:-s, :]], axis=0) + float(s))
        s *= 2
    dopp = jnp.minimum(a, b) + 1.0
    dsq = dopp * dopp

    lane = jax.lax.broadcasted_iota(jnp.int32, (_N, _H), 1)
    oh0 = (lane == p0).astype(jnp.float32)
    gd = jnp.dot(oh0, dsq, preferred_element_type=jnp.float32)
    gh = jnp.dot(oh0, hd, preferred_element_type=jnp.float32)
    g1 = (1.0 - gh) * gd
    g0 = gh * gd

    bb = (lane.astype(jnp.float32) - p1.astype(jnp.float32)) ** 2
    dmin1sq = jnp.min(g1 + bb, axis=1, keepdims=True)
    dmin0sq = jnp.min(g0 + bb, axis=1, keepdims=True)

    outside_frame = (p0 < 0) | (p0 > _H) | (p1 < 0) | (p1 > _W)
    valid = (p0 >= 1) & (p1 >= 1)
    loss_out = jnp.exp(jnp.sqrt(dmin0sq) * (_LN2 / _K2))
    loss_in = jnp.exp(-dmin1sq * (1.0 / _K1))
    lin = jnp.where(outside_frame, 0.0, loss_in)
    lout = jnp.where(outside_frame, 0.0, loss_out)
    out_ref[:, :] = jnp.concatenate(
        [lin, lout, valid.astype(jnp.float32)], axis=1)


_PPW = 16


@functools.partial(
    pl.kernel,
    mesh=plsc.VectorSubcoreMesh(core_axis_name="c", subcore_axis_name="s"),
    out_type=jax.ShapeDtypeStruct((4 * _N,), jnp.float32),
    scratch_types=[
        pltpu.VMEM((_PPW,), jnp.int32),
        pltpu.VMEM((_PPW,), jnp.float32),
        pltpu.SemaphoreType.DMA,
    ],
)
def _nbr_sc(hd_hbm, idx_hbm, out_hbm, idx_v, val_v, sem):
    wid = lax.axis_index("s") * 2 + lax.axis_index("c")
    base = wid * _PPW
    pltpu.sync_copy(idx_hbm.at[pl.ds(base, _PPW)], idx_v)
    pltpu.async_copy(hd_hbm.at[idx_v], val_v, sem).wait()
    pltpu.sync_copy(val_v, out_hbm.at[pl.ds(base, _PPW)])


@jax.jit
def _run(hd_map, prediction):
    p0 = prediction[:, 0]
    p1 = prediction[:, 1]
    rows = jnp.stack([p1 - 1, p1 - 1, p1, p1], axis=1)
    cols = jnp.stack([p0 - 1, p0, p0 - 1, p0], axis=1)
    idx = (jnp.clip(rows, 0, _H - 1) * _W
           + jnp.clip(cols, 0, _W - 1)).reshape(-1)
    vals = _nbr_sc(hd_map.reshape(-1), idx)
    per_pt = pl.pallas_call(
        _road_loss_tc,
        out_shape=jax.ShapeDtypeStruct((_N, 3), jnp.float32),
    )(hd_map, prediction)
    nbr = jnp.sum(vals.reshape(_N, 4), axis=1)
    sel = (per_pt[:, 2] > 0.5) & (nbr > 0.5)
    per = jnp.where(sel, per_pt[:, 1], per_pt[:, 0])
    return jnp.mean(per)


def kernel(hd_map, prediction):
    return _run(hd_map, prediction)


# speedup vs baseline: 4.0549x; 4.0549x over previous
import jax
import jax.numpy as jnp
from jax.experimental import pallas as pl
from jax.experimental.pallas import tpu as pltpu

_H = 512
_W = 512
_N = 128
_G = 4
_SW = _W // _G
_K1 = 21.7
_K2 = 40.0
_BIG = 1.0e4
_LN2 = 0.6931471805599453


def _road_loss_kernel(hd_ref, pred_ref, out_ref, oh_ref, a1_ref, a0_ref,
                      an_ref):
    g = pl.program_id(0)
    hd = hd_ref[:]
    p0 = pred_ref[:, 0:1]
    p1 = pred_ref[:, 1:2]

    @pl.when(g == 0)
    def _build_onehots():
        lane = jax.lax.broadcasted_iota(jnp.int32, (_N, _H), 1)
        oh0 = (lane == p0).astype(jnp.float32)
        oh_ref[0:_N, :] = oh0
        oh_ref[_N:2 * _N, :] = ((lane == p1).astype(jnp.float32)
                                + (lane == p1 - 1).astype(jnp.float32))
        oh_ref[2 * _N:3 * _N, :] = oh0 + (lane == p0 - 1).astype(jnp.float32)

    hdn = jnp.concatenate([hd[1:, :], hd[511:, :]], axis=0)
    e = jnp.where(hd != hdn, 0.0, _BIG)
    a = e
    b = jnp.concatenate([jnp.full((1, _SW), _BIG, jnp.float32),
                         e[:-1, :]], axis=0)
    s = 1
    for _ in range(9):
        pad = jnp.full((s, _SW), _BIG, dtype=jnp.float32)
        a = jnp.minimum(a, jnp.concatenate([a[s:, :], pad], axis=0) + float(s))
        b = jnp.minimum(b, jnp.concatenate([pad, b[:-s, :]], axis=0) + float(s))
        s *= 2
    dopp = jnp.minimum(a, b) + 1.0
    dsq = dopp * dopp

    oh0 = oh_ref[0:_N, :]
    gd = jnp.dot(oh0, dsq, preferred_element_type=jnp.float32)
    gh = jnp.dot(oh0, hd, preferred_element_type=jnp.float32)
    g1 = (1.0 - gh) * gd
    g0 = gh * gd

    gp = jnp.dot(oh_ref[_N:2 * _N, :], hd, preferred_element_type=jnp.float32)
    ohc = oh_ref[2 * _N:3 * _N, pl.ds(g * _SW, _SW)]
    nbrs = jnp.sum(gp * ohc, axis=1, keepdims=True)

    lane_s = jax.lax.broadcasted_iota(jnp.int32, (_N, _SW), 1) + g * _SW
    bb = (lane_s.astype(jnp.float32) - p1.astype(jnp.float32)) ** 2
    m1s = jnp.min(g1 + bb, axis=1, keepdims=True)
    m0s = jnp.min(g0 + bb, axis=1, keepdims=True)

    @pl.when(g == 0)
    def _init_acc():
        a1_ref[:, :] = m1s
        a0_ref[:, :] = m0s
        an_ref[:, :] = nbrs

    @pl.when(g > 0)
    def _acc():
        a1_ref[:, :] = jnp.minimum(a1_ref[:, :], m1s)
        a0_ref[:, :] = jnp.minimum(a0_ref[:, :], m0s)
        an_ref[:, :] = an_ref[:, :] + nbrs

    @pl.when(g == _G - 1)
    def _emit():
        dmin1sq = a1_ref[:, :]
        dmin0sq = a0_ref[:, :]
        nbr = an_ref[:, :]
        outside_frame = (p0 < 0) | (p0 > _H) | (p1 < 0) | (p1 > _W)
        valid = (p0 >= 1) & (p1 >= 1)
        outside_road = valid & (nbr > 0.5)
        loss_out = jnp.exp(jnp.sqrt(dmin0sq) * (_LN2 / _K2))
        loss_in = jnp.exp(-dmin1sq * (1.0 / _K1))
        per = jnp.where(outside_frame, 0.0,
                        jnp.where(outside_road, loss_out, loss_in))
        out_ref[:, :] = jnp.sum(per, axis=0, keepdims=True) * (1.0 / _N)


@jax.jit
def _run(hd_map, prediction):
    return pl.pallas_call(
        _road_loss_kernel,
        grid=(_G,),
        in_specs=[
            pl.BlockSpec((_H, _SW), lambda g: (0, g)),
            pl.BlockSpec((_N, 2), lambda g: (0, 0)),
        ],
        out_specs=pl.BlockSpec((1, 1), lambda g: (0, 0)),
        out_shape=jax.ShapeDtypeStruct((1, 1), jnp.float32),
        scratch_shapes=[
            pltpu.VMEM((3 * _N, _H), jnp.float32),
            pltpu.VMEM((_N, 1), jnp.float32),
            pltpu.VMEM((_N, 1), jnp.float32),
            pltpu.VMEM((_N, 1), jnp.float32),
        ],
    )(hd_map, prediction)


def kernel(hd_map, prediction):
    out = _run(hd_map, prediction)
    return out[0, 0]


# speedup vs baseline: 4.4169x; 1.0893x over previous
import jax
import jax.numpy as jnp
from jax.experimental import pallas as pl
from jax.experimental.pallas import tpu as pltpu

_H = 512
_W = 512
_N = 128
_G = 2
_SW = _W // _G
_K1 = 21.7
_K2 = 40.0
_BIG = 1.0e4
_LN2 = 0.6931471805599453


def _road_loss_kernel(hd_ref, pred_ref, out_ref, oh_ref, a1_ref, a0_ref,
                      an_ref):
    g = pl.program_id(0)
    hd = hd_ref[:]
    p0 = pred_ref[:, 0:1]
    p1 = pred_ref[:, 1:2]

    @pl.when(g == 0)
    def _build_onehots():
        lane = jax.lax.broadcasted_iota(jnp.int32, (_N, _H), 1)
        oh0 = (lane == p0).astype(jnp.float32)
        oh_ref[0:_N, :] = oh0
        oh_ref[_N:2 * _N, :] = ((lane == p1).astype(jnp.float32)
                                + (lane == p1 - 1).astype(jnp.float32))
        oh_ref[2 * _N:3 * _N, :] = oh0 + (lane == p0 - 1).astype(jnp.float32)

    hdn = jnp.concatenate([hd[1:, :], hd[511:, :]], axis=0)
    e = jnp.where(hd != hdn, 0.0, _BIG)
    a = e
    b = jnp.concatenate([jnp.full((1, _SW), _BIG, jnp.float32),
                         e[:-1, :]], axis=0)
    s = 1
    for _ in range(9):
        pad = jnp.full((s, _SW), _BIG, dtype=jnp.float32)
        a = jnp.minimum(a, jnp.concatenate([a[s:, :], pad], axis=0) + float(s))
        b = jnp.minimum(b, jnp.concatenate([pad, b[:-s, :]], axis=0) + float(s))
        s *= 2
    dopp = jnp.minimum(a, b) + 1.0
    dsq = dopp * dopp

    oh0 = oh_ref[0:_N, :]
    gd = jnp.dot(oh0, dsq, preferred_element_type=jnp.float32)
    gh = jnp.dot(oh0, hd, preferred_element_type=jnp.float32)
    g1 = (1.0 - gh) * gd
    g0 = gh * gd

    gp = jnp.dot(oh_ref[_N:2 * _N, :], hd, preferred_element_type=jnp.float32)
    ohc = oh_ref[2 * _N:3 * _N, pl.ds(g * _SW, _SW)]
    nbrs = jnp.sum(gp * ohc, axis=1, keepdims=True)

    lane_s = jax.lax.broadcasted_iota(jnp.int32, (_N, _SW), 1) + g * _SW
    bb = (lane_s.astype(jnp.float32) - p1.astype(jnp.float32)) ** 2
    m1s = jnp.min(g1 + bb, axis=1, keepdims=True)
    m0s = jnp.min(g0 + bb, axis=1, keepdims=True)

    @pl.when(g == 0)
    def _init_acc():
        a1_ref[:, :] = m1s
        a0_ref[:, :] = m0s
        an_ref[:, :] = nbrs

    @pl.when(g > 0)
    def _acc():
        a1_ref[:, :] = jnp.minimum(a1_ref[:, :], m1s)
        a0_ref[:, :] = jnp.minimum(a0_ref[:, :], m0s)
        an_ref[:, :] = an_ref[:, :] + nbrs

    @pl.when(g == _G - 1)
    def _emit():
        dmin1sq = a1_ref[:, :]
        dmin0sq = a0_ref[:, :]
        nbr = an_ref[:, :]
        outside_frame = (p0 < 0) | (p0 > _H) | (p1 < 0) | (p1 > _W)
        valid = (p0 >= 1) & (p1 >= 1)
        outside_road = valid & (nbr > 0.5)
        loss_out = jnp.exp(jnp.sqrt(dmin0sq) * (_LN2 / _K2))
        loss_in = jnp.exp(-dmin1sq * (1.0 / _K1))
        per = jnp.where(outside_frame, 0.0,
                        jnp.where(outside_road, loss_out, loss_in))
        out_ref[:, :] = jnp.sum(per, axis=0, keepdims=True) * (1.0 / _N)


@jax.jit
def _run(hd_map, prediction):
    return pl.pallas_call(
        _road_loss_kernel,
        grid=(_G,),
        in_specs=[
            pl.BlockSpec((_H, _SW), lambda g: (0, g)),
            pl.BlockSpec((_N, 2), lambda g: (0, 0)),
        ],
        out_specs=pl.BlockSpec((1, 1), lambda g: (0, 0)),
        out_shape=jax.ShapeDtypeStruct((1, 1), jnp.float32),
        scratch_shapes=[
            pltpu.VMEM((3 * _N, _H), jnp.float32),
            pltpu.VMEM((_N, 1), jnp.float32),
            pltpu.VMEM((_N, 1), jnp.float32),
            pltpu.VMEM((_N, 1), jnp.float32),
        ],
    )(hd_map, prediction)


def kernel(hd_map, prediction):
    out = _run(hd_map, prediction)
    return out[0, 0]
